# Initial kernel scaffold; baseline (speedup 1.0000x reference)
import jax
import jax.numpy as jnp
from jax.experimental import pallas as pl


def kernel(x, edge_index, W_in, b_in, W1, b1, W2, b2, W3, b3, W_cls, b_cls):
    raise NotImplementedError("write your pallas kernel here")


# speedup vs baseline: 8.9814x; 8.9814x over previous
import jax
import jax.numpy as jnp
from jax import lax
from jax.experimental import pallas as pl
from jax.experimental.pallas import tpu as pltpu
from jax.experimental.pallas import tpu_sc as plsc

_f32 = jnp.float32

N = 10000
D = 128
DOUT = 40
NC = 2
NS = 16
CHUNK = 128
RPT = 624
RPT_LAST = N - 15 * RPT
JUNK = N
ACC_ROWS = N + 16
DEG_PAD = 10240
DRT = DEG_PAD // NS


def _sc_degree(dstp, e_pad):
    ept = e_pad // (NC * NS)
    nch = ept // CHUNK

    def body(dst_hbm, out_hbm, dst_v, ones_v, zbuf_v, deg_sh):
        cid = lax.axis_index("c")
        sid = lax.axis_index("s")
        for i in range(DRT // 16):
            zbuf_v[pl.ds(i * 16, 16)] = jnp.zeros((16,), _f32)
        for i in range(CHUNK // 16):
            ones_v[pl.ds(i * 16, 16)] = jnp.ones((16,), _f32)
        pltpu.sync_copy(zbuf_v, deg_sh.at[pl.ds(sid * DRT, DRT)])
        plsc.subcore_barrier()
        ebase = cid * (e_pad // NC) + sid * ept

        def step(i, c):
            b = ebase + i * CHUNK
            pltpu.sync_copy(dst_hbm.at[pl.ds(b, CHUNK)], dst_v)
            pltpu.sync_copy(ones_v, deg_sh.at[dst_v], add=True)
            return c

        lax.fori_loop(0, nch, step, 0)
        plsc.subcore_barrier()
        pltpu.sync_copy(deg_sh.at[pl.ds(sid * DRT, DRT)],
                        out_hbm.at[cid, pl.ds(sid * DRT, DRT)])

    return pl.kernel(
        body,
        out_type=jax.ShapeDtypeStruct((NC, DEG_PAD), _f32),
        mesh=plsc.VectorSubcoreMesh(core_axis_name="c", subcore_axis_name="s"),
        scratch_types=[
            pltpu.VMEM((CHUNK,), jnp.int32),
            pltpu.VMEM((CHUNK,), _f32),
            pltpu.VMEM((DRT,), _f32),
            pltpu.VMEM_SHARED((DEG_PAD,), _f32),
        ],
    )(dstp)


def _sc_scatter(z, srcp, dstp, zero_rows, e_pad):
    ept = e_pad // (NC * NS)
    nch = ept // CHUNK

    def body(z_hbm, src_hbm, dst_hbm, zero_hbm, out_hbm,
             src_v, dst_v, rows_v, acc_sh, sem):
        cid = lax.axis_index("c")
        sid = lax.axis_index("s")
        rb = sid * RPT

        @pl.when(jnp.logical_and(cid == 0, sid < NS - 1))
        def _():
            pltpu.sync_copy(z_hbm.at[pl.ds(rb, RPT)], acc_sh.at[pl.ds(rb, RPT)])

        @pl.when(jnp.logical_and(cid == 0, sid == NS - 1))
        def _():
            pltpu.sync_copy(z_hbm.at[pl.ds(rb, RPT_LAST)],
                            acc_sh.at[pl.ds(rb, RPT_LAST)])

        @pl.when(jnp.logical_and(cid != 0, sid < NS - 1))
        def _():
            pltpu.sync_copy(zero_hbm.at[pl.ds(0, RPT)],
                            acc_sh.at[pl.ds(rb, RPT)])

        @pl.when(jnp.logical_and(cid != 0, sid == NS - 1))
        def _():
            pltpu.sync_copy(zero_hbm, acc_sh.at[pl.ds(rb, RPT_LAST)])

        plsc.subcore_barrier()
        ebase = cid * (e_pad // NC) + sid * ept

        def step(i, c):
            b = ebase + i * CHUNK
            pltpu.sync_copy(src_hbm.at[pl.ds(b, CHUNK)], src_v)
            pltpu.sync_copy(dst_hbm.at[pl.ds(b, CHUNK)], dst_v)
            pltpu.async_copy(z_hbm.at[src_v], rows_v, sem).wait()
            pltpu.sync_copy(rows_v, acc_sh.at[dst_v], add=True)
            return c

        lax.fori_loop(0, nch, step, 0)
        plsc.subcore_barrier()

        @pl.when(sid < NS - 1)
        def _():
            pltpu.sync_copy(acc_sh.at[pl.ds(rb, RPT)],
                            out_hbm.at[cid, pl.ds(rb, RPT)])

        @pl.when(sid == NS - 1)
        def _():
            pltpu.sync_copy(acc_sh.at[pl.ds(rb, RPT_LAST)],
                            out_hbm.at[cid, pl.ds(rb, RPT_LAST)])

    return pl.kernel(
        body,
        out_type=jax.ShapeDtypeStruct((NC, N, D), _f32),
        mesh=plsc.VectorSubcoreMesh(core_axis_name="c", subcore_axis_name="s"),
        scratch_types=[
            pltpu.VMEM((CHUNK,), jnp.int32),
            pltpu.VMEM((CHUNK,), jnp.int32),
            pltpu.VMEM((CHUNK, D), _f32),
            pltpu.VMEM_SHARED((ACC_ROWS, D), _f32),
            pltpu.SemaphoreType.DMA,
        ],
    )(z, srcp, dstp, zero_rows)


_R = 1000


def _dot(a, b):
    return lax.dot_general(a, b, (((1,), (0,)), ((), ())),
                           precision=lax.Precision.HIGHEST,
                           preferred_element_type=_f32)


def _tc_stage_in(x, W_in, b_in, W1, deg2):
    def body(x_ref, win_ref, bin_ref, w1_ref, deg_ref, z1_ref, dinv_ref):
        h = jnp.maximum(_dot(x_ref[...], win_ref[...]) + bin_ref[...], 0.0)
        dinv = lax.rsqrt(deg_ref[:, 0:1] + deg_ref[:, 1:2] + 1.0)
        z1_ref[...] = _dot(h, w1_ref[...]) * dinv
        dinv_ref[...] = dinv

    return pl.pallas_call(
        body,
        grid=(N // _R,),
        in_specs=[
            pl.BlockSpec((_R, D), lambda i: (i, 0)),
            pl.BlockSpec((D, D), lambda i: (0, 0)),
            pl.BlockSpec((1, D), lambda i: (0, 0)),
            pl.BlockSpec((D, D), lambda i: (0, 0)),
            pl.BlockSpec((_R, 2), lambda i: (i, 0)),
        ],
        out_specs=[
            pl.BlockSpec((_R, D), lambda i: (i, 0)),
            pl.BlockSpec((_R, 1), lambda i: (i, 0)),
        ],
        out_shape=[
            jax.ShapeDtypeStruct((N, D), _f32),
            jax.ShapeDtypeStruct((N, 1), _f32),
        ],
    )(x, W_in, b_in.reshape(1, D), W1, deg2)


def _tc_stage_mid(agg, dinv, b, Wn):
    def body(agg_ref, dinv_ref, b_ref, w_ref, h_ref, zn_ref):
        dv = dinv_ref[...]
        t = jnp.tanh((agg_ref[0] + agg_ref[1]) * dv + b_ref[...])
        h_ref[...] = t
        zn_ref[...] = _dot(t, w_ref[...]) * dv

    return pl.pallas_call(
        body,
        grid=(N // _R,),
        in_specs=[
            pl.BlockSpec((NC, _R, D), lambda i: (0, i, 0)),
            pl.BlockSpec((_R, 1), lambda i: (i, 0)),
            pl.BlockSpec((1, D), lambda i: (0, 0)),
            pl.BlockSpec((D, D), lambda i: (0, 0)),
        ],
        out_specs=[
            pl.BlockSpec((_R, D), lambda i: (i, 0)),
            pl.BlockSpec((_R, D), lambda i: (i, 0)),
        ],
        out_shape=[
            jax.ShapeDtypeStruct((N, D), _f32),
            jax.ShapeDtypeStruct((N, D), _f32),
        ],
    )(agg, dinv, b.reshape(1, D), Wn)


def _tc_stage_out(agg, dinv, b3, W_cls, b_cls):
    def body(agg_ref, dinv_ref, b3_ref, wc_ref, bc_ref, h_ref, y_ref):
        t = jnp.tanh((agg_ref[0] + agg_ref[1]) * dinv_ref[...] + b3_ref[...])
        h_ref[...] = t
        y_ref[...] = _dot(t, wc_ref[...]) + bc_ref[...]

    return pl.pallas_call(
        body,
        grid=(N // _R,),
        in_specs=[
            pl.BlockSpec((NC, _R, D), lambda i: (0, i, 0)),
            pl.BlockSpec((_R, 1), lambda i: (i, 0)),
            pl.BlockSpec((1, D), lambda i: (0, 0)),
            pl.BlockSpec((D, DOUT), lambda i: (0, 0)),
            pl.BlockSpec((1, DOUT), lambda i: (0, 0)),
        ],
        out_specs=[
            pl.BlockSpec((_R, D), lambda i: (i, 0)),
            pl.BlockSpec((_R, DOUT), lambda i: (i, 0)),
        ],
        out_shape=[
            jax.ShapeDtypeStruct((N, D), _f32),
            jax.ShapeDtypeStruct((N, DOUT), _f32),
        ],
    )(agg, dinv, b3.reshape(1, D), W_cls, b_cls.reshape(1, DOUT))


def kernel(x, edge_index, W_in, b_in, W1, b1, W2, b2, W3, b3, W_cls, b_cls):
    src = edge_index[0]
    dst = edge_index[1]
    e = src.shape[0]
    grp = NC * NS * CHUNK
    e_pad = ((e + grp - 1) // grp) * grp
    pad = e_pad - e
    srcp = jnp.concatenate([src, jnp.zeros((pad,), src.dtype)])
    dstp = jnp.concatenate([dst, jnp.full((pad,), JUNK, dst.dtype)])
    zero_rows = jnp.zeros((RPT_LAST, D), _f32)

    deg = _sc_degree(dstp, e_pad)
    deg2 = deg[:, :N].T

    z1, dinv = _tc_stage_in(x, W_in, b_in, W1, deg2)
    agg1 = _sc_scatter(z1, srcp, dstp, zero_rows, e_pad)
    h1, z2 = _tc_stage_mid(agg1, dinv, b1, W2)
    agg2 = _sc_scatter(z2, srcp, dstp, zero_rows, e_pad)
    h2, z3 = _tc_stage_mid(agg2, dinv, b2, W3)
    agg3 = _sc_scatter(z3, srcp, dstp, zero_rows, e_pad)
    h3, y = _tc_stage_out(agg3, dinv, b3, W_cls, b_cls)
    return (h1, h2, h3, y)
